# baseline (speedup 1.0000x reference)
import functools

import jax
import jax.numpy as jnp
from jax import lax
from jax.experimental import pallas as pl
from jax.experimental.pallas import tpu as pltpu
from jax.experimental.pallas import tpu_sc as plsc

_INFO = plsc.get_sparse_core_info()
_NC = _INFO.num_cores
_NS = _INFO.num_subcores
_NW = _NC * _NS

_CHUNK = 64
_NBUF = 2


def _build_gather(R, C, V, D):
    B = R * C
    assert B % _NW == 0
    b_per_w = B // _NW
    assert C % b_per_w == 0
    w_per_r = C // b_per_w
    assert b_per_w % _CHUNK == 0
    n_chunks = b_per_w // _CHUNK

    mesh = plsc.VectorSubcoreMesh(core_axis_name="c", subcore_axis_name="s")

    @functools.partial(
        pl.kernel,
        mesh=mesh,
        out_type=jax.ShapeDtypeStruct((B, D), jnp.float32),
        scratch_types=[
            pltpu.VMEM((b_per_w,), jnp.int32),
            pltpu.VMEM((_NBUF, _CHUNK, D), jnp.float32),
        ]
        + [pltpu.SemaphoreType.DMA] * (2 * _NBUF),
    )
    def k(idx_hbm, table_hbm, out_hbm, idx_v, rows_v, *sems):
        gsems, osems = sems[:_NBUF], sems[_NBUF:]
        wid = lax.axis_index("s") * _NC + lax.axis_index("c")
        base = wid * b_per_w
        pltpu.sync_copy(
            idx_hbm.at[wid // w_per_r, pl.ds((wid % w_per_r) * b_per_w, b_per_w)],
            idx_v,
        )
        gcp = [None] * _NBUF
        ocp = [None] * _NBUF
        for c in range(n_chunks):
            b = c % _NBUF
            if c >= _NBUF:
                ocp[b].wait()
            gcp[b] = pltpu.async_copy(
                table_hbm.at[idx_v.at[pl.ds(c * _CHUNK, _CHUNK)]],
                rows_v.at[b],
                gsems[b],
            )
            if c >= 1:
                pb = (c - 1) % _NBUF
                gcp[pb].wait()
                ocp[pb] = pltpu.async_copy(
                    rows_v.at[pb],
                    out_hbm.at[pl.ds(base + (c - 1) * _CHUNK, _CHUNK)],
                    osems[pb],
                )
        lb = (n_chunks - 1) % _NBUF
        gcp[lb].wait()
        ocp[lb] = pltpu.async_copy(
            rows_v.at[lb],
            out_hbm.at[pl.ds(base + (n_chunks - 1) * _CHUNK, _CHUNK)],
            osems[lb],
        )
        for b in range(min(_NBUF, n_chunks)):
            ocp[(n_chunks - 1 - b) % _NBUF].wait()

    return k


def kernel(x, table):
    R, C = x.shape
    V, D = table.shape
    out = _build_gather(R, C, V, D)(x, table)
    return out.reshape(R, C, D)


# speedup vs baseline: 1.0300x; 1.0300x over previous
import functools

import jax
import jax.numpy as jnp
from jax import lax
from jax.experimental import pallas as pl
from jax.experimental.pallas import tpu as pltpu
from jax.experimental.pallas import tpu_sc as plsc

_INFO = plsc.get_sparse_core_info()
_NC = _INFO.num_cores
_NS = _INFO.num_subcores
_NW = _NC * _NS

_CHUNKS = (16, 48, 64, 64, 48, 16)
_BUFCAP = 64
_NBUF = 2


def _build_gather(R, C, V, D):
    B = R * C
    assert B % _NW == 0
    b_per_w = B // _NW
    assert C % b_per_w == 0
    w_per_r = C // b_per_w
    assert sum(_CHUNKS) == b_per_w
    n_chunks = len(_CHUNKS)
    starts = [sum(_CHUNKS[:i]) for i in range(n_chunks)]

    mesh = plsc.VectorSubcoreMesh(core_axis_name="c", subcore_axis_name="s")

    @functools.partial(
        pl.kernel,
        mesh=mesh,
        out_type=jax.ShapeDtypeStruct((B, D), jnp.float32),
        scratch_types=[
            pltpu.VMEM((b_per_w,), jnp.int32),
            pltpu.VMEM((_NBUF, _BUFCAP, D), jnp.float32),
        ]
        + [pltpu.SemaphoreType.DMA] * (2 * _NBUF),
    )
    def k(idx_hbm, table_hbm, out_hbm, idx_v, rows_v, *sems):
        gsems, osems = sems[:_NBUF], sems[_NBUF:]
        wid = lax.axis_index("s") * _NC + lax.axis_index("c")
        base = wid * b_per_w
        pltpu.sync_copy(
            idx_hbm.at[wid // w_per_r, pl.ds((wid % w_per_r) * b_per_w, b_per_w)],
            idx_v,
        )
        gcp = [None] * _NBUF
        ocp = [None] * _NBUF
        for c in range(n_chunks):
            b = c % _NBUF
            cs = _CHUNKS[c]
            if c >= _NBUF:
                ocp[b].wait()
            gcp[b] = pltpu.async_copy(
                table_hbm.at[idx_v.at[pl.ds(starts[c], cs)]],
                rows_v.at[b, pl.ds(0, cs)],
                gsems[b],
            )
            if c >= 1:
                pb = (c - 1) % _NBUF
                gcp[pb].wait()
                ocp[pb] = pltpu.async_copy(
                    rows_v.at[pb, pl.ds(0, _CHUNKS[c - 1])],
                    out_hbm.at[pl.ds(base + starts[c - 1], _CHUNKS[c - 1])],
                    osems[pb],
                )
        lb = (n_chunks - 1) % _NBUF
        gcp[lb].wait()
        ocp[lb] = pltpu.async_copy(
            rows_v.at[lb, pl.ds(0, _CHUNKS[n_chunks - 1])],
            out_hbm.at[pl.ds(base + starts[n_chunks - 1], _CHUNKS[n_chunks - 1])],
            osems[lb],
        )
        for b in range(min(_NBUF, n_chunks)):
            ocp[(n_chunks - 1 - b) % _NBUF].wait()

    return k


def kernel(x, table):
    R, C = x.shape
    V, D = table.shape
    out = _build_gather(R, C, V, D)(x, table)
    return out.reshape(R, C, D)
